# baseline (speedup 1.0000x reference)
import functools

import jax
import jax.numpy as jnp
from jax import lax
from jax.experimental import pallas as pl
from jax.experimental.pallas import tpu as pltpu
from jax.experimental.pallas import tpu_sc as plsc

BATCH = 16384
STIM = 128
EMB = 16

_NC = 2
_NS = 16
_NW = _NC * _NS
_CHUNK = 128
_N_CHUNKS = BATCH // _CHUNK
_CH_PER_W = _N_CHUNKS // _NW


def _sc_gather(table, ids2d):
    mesh = plsc.VectorSubcoreMesh(core_axis_name="c", subcore_axis_name="s")

    @functools.partial(
        pl.kernel,
        mesh=mesh,
        compiler_params=pltpu.CompilerParams(use_tc_tiling_on_sc=False),
        out_type=jax.ShapeDtypeStruct((_N_CHUNKS, _CHUNK, EMB), jnp.float32),
        scratch_types=[
            pltpu.VMEM((_CH_PER_W, _CHUNK), jnp.int32),
            pltpu.VMEM((_CH_PER_W, _CHUNK, EMB), jnp.float32),
            pltpu.SemaphoreType.DMA,
        ],
    )
    def gath(table_hbm, ids_hbm, out_hbm, idx_v, rows_v, sem):
        wid = lax.axis_index("s") * _NC + lax.axis_index("c")
        base = wid * _CH_PER_W
        pltpu.sync_copy(ids_hbm.at[pl.ds(base, _CH_PER_W)], idx_v)
        copies = [
            pltpu.async_copy(table_hbm.at[idx_v.at[j]], rows_v.at[j], sem)
            for j in range(_CH_PER_W)
        ]
        for c in copies:
            c.wait()
        pltpu.sync_copy(rows_v, out_hbm.at[pl.ds(base, _CH_PER_W)])

    return gath(table, ids2d)


def _tc_fused(x, e, wx, we, b2):
    blk = 2048

    def body(b_ref, x_ref, e_ref, wx_ref, we_ref, o_ref):
        acc = jnp.dot(x_ref[...], wx_ref[...], preferred_element_type=jnp.float32)
        acc = acc + jnp.dot(e_ref[...], we_ref[...], preferred_element_type=jnp.float32)
        o_ref[...] = jax.nn.sigmoid(acc + b_ref[0])

    return pl.pallas_call(
        body,
        grid=(BATCH // blk,),
        in_specs=[
            pl.BlockSpec(memory_space=pltpu.SMEM),
            pl.BlockSpec((blk, STIM), lambda i: (i, 0)),
            pl.BlockSpec((blk, EMB), lambda i: (i, 0)),
            pl.BlockSpec((STIM, 1), lambda i: (0, 0)),
            pl.BlockSpec((EMB, 1), lambda i: (0, 0)),
        ],
        out_specs=pl.BlockSpec((blk, 1), lambda i: (i, 0)),
        out_shape=jax.ShapeDtypeStruct((BATCH, 1), jnp.float32),
    )(b2, x, e, wx, we)


def kernel(X_stim, neuron_ids, embed_table, W, b):
    ids2d = neuron_ids.astype(jnp.int32).reshape(_N_CHUNKS, _CHUNK)
    e = _sc_gather(embed_table, ids2d).reshape(BATCH, EMB)
    wx = W[0, :STIM].reshape(STIM, 1)
    we = W[0, STIM:].reshape(EMB, 1)
    probs = _tc_fused(X_stim, e, wx, we, b)
    return probs.reshape(BATCH)


# speedup vs baseline: 1.0293x; 1.0293x over previous
import functools

import jax
import jax.numpy as jnp
from jax import lax
from jax.experimental import pallas as pl
from jax.experimental.pallas import tpu as pltpu
from jax.experimental.pallas import tpu_sc as plsc

BATCH = 16384
STIM = 128
EMB = 16

_NC = 2
_NS = 16
_NW = _NC * _NS
_CHUNK = 128
_N_CHUNKS = BATCH // _CHUNK
_CH_PER_W = _N_CHUNKS // _NW
_GRP = _CHUNK // 16
_TROWS = (1000000 * EMB) // STIM


def _sc_embed_dot(table128, ids2d, we):
    mesh = plsc.VectorSubcoreMesh(core_axis_name="c", subcore_axis_name="s")

    @functools.partial(
        pl.kernel,
        mesh=mesh,
        compiler_params=pltpu.CompilerParams(needs_layout_passes=False),
        out_type=jax.ShapeDtypeStruct((_N_CHUNKS, _CHUNK), jnp.float32),
        scratch_types=[
            pltpu.VMEM((_CH_PER_W, _CHUNK), jnp.int32),
            pltpu.VMEM((_CH_PER_W, _CHUNK), jnp.int32),
            pltpu.VMEM((_CH_PER_W, _CHUNK, STIM), jnp.float32),
            pltpu.VMEM((EMB,), jnp.float32),
            pltpu.VMEM((_CH_PER_W, _CHUNK), jnp.float32),
            pltpu.SemaphoreType.DMA,
        ],
    )
    def body(table_hbm, ids_hbm, we_hbm, out_hbm, idx_raw, idx_hi, rows_v,
             we_v, out_v, sem):
        wid = lax.axis_index("s") * _NC + lax.axis_index("c")
        base = wid * _CH_PER_W
        pltpu.sync_copy(ids_hbm.at[pl.ds(base, _CH_PER_W)], idx_raw)
        pltpu.sync_copy(we_hbm, we_v)
        for j in range(_CH_PER_W):
            for g in range(_GRP):
                v = idx_raw[j, pl.ds(g * 16, 16)]
                idx_hi[j, pl.ds(g * 16, 16)] = lax.shift_right_logical(v, 3)
        copies = [
            pltpu.async_copy(table_hbm.at[idx_hi.at[j]], rows_v.at[j], sem)
            for j in range(_CH_PER_W)
        ]
        wvec = we_v[...]
        wed = [wvec[d] for d in range(EMB)]
        iota = lax.iota(jnp.int32, 16)
        for j in range(_CH_PER_W):
            copies[j].wait()
            jv = jnp.full((16,), j, jnp.int32)
            for g in range(_GRP):
                v = idx_raw[j, pl.ds(g * 16, 16)]
                rem16 = lax.shift_left(v & 7, 4)
                i1 = g * 16 + iota
                col = plsc.load_gather(rows_v, [jv, i1, rem16])
                acc = col * wed[0]
                for d in range(1, EMB):
                    col = plsc.load_gather(rows_v, [jv, i1, rem16 + d])
                    acc = acc + col * wed[d]
                out_v[j, pl.ds(g * 16, 16)] = acc
        pltpu.sync_copy(out_v, out_hbm.at[pl.ds(base, _CH_PER_W)])

    return body(table128, ids2d, we)


def _tc_matvec(x3, wx):
    blk = 16

    def body(x_ref, w_ref, o_ref):
        prod = x_ref[...] * w_ref[...][None]
        o_ref[...] = jnp.sum(prod, axis=-1)

    return pl.pallas_call(
        body,
        grid=(128 // blk,),
        in_specs=[
            pl.BlockSpec((blk, 128, STIM), lambda i: (i, 0, 0)),
            pl.BlockSpec((1, STIM), lambda i: (0, 0)),
        ],
        out_specs=pl.BlockSpec((blk, 128), lambda i: (i, 0)),
        out_shape=jax.ShapeDtypeStruct((128, 128), jnp.float32),
    )(x3, wx)


def _tc_combine(stim, emb, b):
    def body(b_ref, s_ref, e_ref, o_ref):
        o_ref[...] = jax.nn.sigmoid(s_ref[...] + e_ref[...] + b_ref[0])

    return pl.pallas_call(
        body,
        in_specs=[
            pl.BlockSpec(memory_space=pltpu.SMEM),
            pl.BlockSpec((128, 128), lambda: (0, 0)),
            pl.BlockSpec((128, 128), lambda: (0, 0)),
        ],
        out_specs=pl.BlockSpec((128, 128), lambda: (0, 0)),
        out_shape=jax.ShapeDtypeStruct((128, 128), jnp.float32),
    )(b, stim, emb)


def kernel(X_stim, neuron_ids, embed_table, W, b):
    ids2d = neuron_ids.astype(jnp.int32).reshape(_N_CHUNKS, _CHUNK)
    table128 = embed_table.reshape(_TROWS, STIM)
    we = W[0, STIM:]
    emb = _sc_embed_dot(table128, ids2d, we)
    x3 = X_stim.reshape(128, 128, STIM)
    wx = W[0, :STIM].reshape(1, STIM)
    stim = _tc_matvec(x3, wx)
    probs = _tc_combine(stim, emb, b)
    return probs.reshape(BATCH)


# speedup vs baseline: 6.0314x; 5.8598x over previous
import functools

import jax
import jax.numpy as jnp
from jax import lax
from jax.experimental import pallas as pl
from jax.experimental.pallas import tpu as pltpu
from jax.experimental.pallas import tpu_sc as plsc

BATCH = 16384
STIM = 128
EMB = 16
NROWS = 1000000

_NC = 2
_NS = 16
_NW = _NC * _NS
_CHUNK = 128
_N_CHUNKS = BATCH // _CHUNK
_CH_PER_W = _N_CHUNKS // _NW
_GRP = _CHUNK // 16

_SBLK = 16384
_SGRID = 62
_SROWS = _SGRID * (_SBLK // 128)


def _tc_plane_sum(table_t, we):
    def body(t_ref, w_ref, o_ref):
        prod = t_ref[...] * w_ref[...]
        red = jnp.sum(prod, axis=0)
        o_ref[...] = red.reshape(_SBLK // 128, 128)

    return pl.pallas_call(
        body,
        grid=(_SGRID,),
        in_specs=[
            pl.BlockSpec((EMB, _SBLK), lambda i: (0, i)),
            pl.BlockSpec((EMB, 1), lambda i: (0, 0)),
        ],
        out_specs=pl.BlockSpec((_SBLK // 128, 128), lambda i: (i, 0)),
        out_shape=jax.ShapeDtypeStruct((_SROWS, 128), jnp.float32),
    )(table_t, we)


def _sc_row_pick(s2d, ids2d):
    mesh = plsc.VectorSubcoreMesh(core_axis_name="c", subcore_axis_name="s")

    @functools.partial(
        pl.kernel,
        mesh=mesh,
        compiler_params=pltpu.CompilerParams(needs_layout_passes=False),
        out_type=jax.ShapeDtypeStruct((_N_CHUNKS, _CHUNK), jnp.float32),
        scratch_types=[
            pltpu.VMEM((_CH_PER_W, _CHUNK), jnp.int32),
            pltpu.VMEM((_CH_PER_W, _CHUNK), jnp.int32),
            pltpu.VMEM((_CH_PER_W, _CHUNK, 128), jnp.float32),
            pltpu.VMEM((_CH_PER_W, _CHUNK), jnp.float32),
            pltpu.SemaphoreType.DMA,
        ],
    )
    def body(s_hbm, ids_hbm, out_hbm, idx_raw, idx_hi, rows_v, out_v, sem):
        wid = lax.axis_index("s") * _NC + lax.axis_index("c")
        base = wid * _CH_PER_W
        pltpu.sync_copy(ids_hbm.at[pl.ds(base, _CH_PER_W)], idx_raw)
        for j in range(_CH_PER_W):
            for g in range(_GRP):
                v = idx_raw[j, pl.ds(g * 16, 16)]
                idx_hi[j, pl.ds(g * 16, 16)] = lax.shift_right_logical(v, 7)
        copies = [
            pltpu.async_copy(s_hbm.at[idx_hi.at[j]], rows_v.at[j], sem)
            for j in range(_CH_PER_W)
        ]
        iota = lax.iota(jnp.int32, 16)
        for j in range(_CH_PER_W):
            copies[j].wait()
            jv = jnp.full((16,), j, jnp.int32)
            for g in range(_GRP):
                v = idx_raw[j, pl.ds(g * 16, 16)]
                lane = v & 127
                i1 = g * 16 + iota
                out_v[j, pl.ds(g * 16, 16)] = plsc.load_gather(
                    rows_v, [jv, i1, lane])
        pltpu.sync_copy(out_v, out_hbm.at[pl.ds(base, _CH_PER_W)])

    return body(s2d, ids2d)


def _tc_matvec(x3, wx):
    blk = 16

    def body(x_ref, w_ref, o_ref):
        prod = x_ref[...] * w_ref[...][None]
        o_ref[...] = jnp.sum(prod, axis=-1)

    return pl.pallas_call(
        body,
        grid=(128 // blk,),
        in_specs=[
            pl.BlockSpec((blk, 128, STIM), lambda i: (i, 0, 0)),
            pl.BlockSpec((1, STIM), lambda i: (0, 0)),
        ],
        out_specs=pl.BlockSpec((blk, 128), lambda i: (i, 0)),
        out_shape=jax.ShapeDtypeStruct((128, 128), jnp.float32),
    )(x3, wx)


def _tc_combine(stim, emb, b):
    def body(b_ref, s_ref, e_ref, o_ref):
        o_ref[...] = jax.nn.sigmoid(s_ref[...] + e_ref[...] + b_ref[0])

    return pl.pallas_call(
        body,
        in_specs=[
            pl.BlockSpec(memory_space=pltpu.SMEM),
            pl.BlockSpec((128, 128), lambda: (0, 0)),
            pl.BlockSpec((128, 128), lambda: (0, 0)),
        ],
        out_specs=pl.BlockSpec((128, 128), lambda: (0, 0)),
        out_shape=jax.ShapeDtypeStruct((128, 128), jnp.float32),
    )(b, stim, emb)


def kernel(X_stim, neuron_ids, embed_table, W, b):
    ids2d = neuron_ids.astype(jnp.int32).reshape(_N_CHUNKS, _CHUNK)
    table_t = jnp.transpose(embed_table)
    we = W[0, STIM:].reshape(EMB, 1)
    s2d = _tc_plane_sum(table_t, we)
    emb = _sc_row_pick(s2d, ids2d)
    x3 = X_stim.reshape(128, 128, STIM)
    wx = W[0, :STIM].reshape(1, STIM)
    stim = _tc_matvec(x3, wx)
    probs = _tc_combine(stim, emb, b)
    return probs.reshape(BATCH)


# speedup vs baseline: 7.5508x; 1.2519x over previous
import functools

import jax
import jax.numpy as jnp
from jax import lax
from jax.experimental import pallas as pl
from jax.experimental.pallas import tpu as pltpu
from jax.experimental.pallas import tpu_sc as plsc

BATCH = 16384
STIM = 128
EMB = 16
NROWS = 1000000

_NC = 2
_NS = 16
_NW = _NC * _NS
_CHUNK = 128
_N_CHUNKS = BATCH // _CHUNK
_CH_PER_W = _N_CHUNKS // _NW
_GRP = _CHUNK // 16

_SBLK = 32768
_SGRID = 31
_SROWS = _SGRID * (_SBLK // 128)


def _tc_plane_sum(table_t, we):
    def body(t_ref, w_ref, o_ref):
        prod = t_ref[...] * w_ref[...]
        red = jnp.sum(prod, axis=0)
        o_ref[...] = red.reshape(_SBLK // 128, 128)

    return pl.pallas_call(
        body,
        grid=(_SGRID,),
        in_specs=[
            pl.BlockSpec((EMB, _SBLK), lambda i: (0, i)),
            pl.BlockSpec((EMB, 1), lambda i: (0, 0)),
        ],
        out_specs=pl.BlockSpec((_SBLK // 128, 128), lambda i: (i, 0)),
        out_shape=jax.ShapeDtypeStruct((_SROWS, 128), jnp.float32),
    )(table_t, we)


def _sc_row_pick(s2d, ids2d):
    mesh = plsc.VectorSubcoreMesh(core_axis_name="c", subcore_axis_name="s")

    @functools.partial(
        pl.kernel,
        mesh=mesh,
        compiler_params=pltpu.CompilerParams(needs_layout_passes=False),
        out_type=jax.ShapeDtypeStruct((_N_CHUNKS, _CHUNK), jnp.float32),
        scratch_types=[
            pltpu.VMEM((_CH_PER_W, _CHUNK), jnp.int32),
            pltpu.VMEM((_CH_PER_W, _CHUNK), jnp.int32),
            pltpu.VMEM((_CH_PER_W, _CHUNK, 128), jnp.float32),
            pltpu.VMEM((_CH_PER_W, _CHUNK), jnp.float32),
            pltpu.SemaphoreType.DMA,
        ],
    )
    def body(s_hbm, ids_hbm, out_hbm, idx_raw, idx_hi, rows_v, out_v, sem):
        wid = lax.axis_index("s") * _NC + lax.axis_index("c")
        base = wid * _CH_PER_W
        pltpu.sync_copy(ids_hbm.at[pl.ds(base, _CH_PER_W)], idx_raw)
        for j in range(_CH_PER_W):
            for g in range(_GRP):
                v = idx_raw[j, pl.ds(g * 16, 16)]
                idx_hi[j, pl.ds(g * 16, 16)] = lax.shift_right_logical(v, 7)
        copies = [
            pltpu.async_copy(s_hbm.at[idx_hi.at[j]], rows_v.at[j], sem)
            for j in range(_CH_PER_W)
        ]
        iota = lax.iota(jnp.int32, 16)
        for j in range(_CH_PER_W):
            copies[j].wait()
            jv = jnp.full((16,), j, jnp.int32)
            for g in range(_GRP):
                v = idx_raw[j, pl.ds(g * 16, 16)]
                lane = v & 127
                i1 = g * 16 + iota
                out_v[j, pl.ds(g * 16, 16)] = plsc.load_gather(
                    rows_v, [jv, i1, lane])
        pltpu.sync_copy(out_v, out_hbm.at[pl.ds(base, _CH_PER_W)])

    return body(s2d, ids2d)


def _tc_matvec(x3, wx):
    blk = 32

    def body(x_ref, w_ref, o_ref):
        prod = x_ref[...] * w_ref[...][None]
        o_ref[...] = jnp.sum(prod, axis=-1)

    return pl.pallas_call(
        body,
        grid=(128 // blk,),
        in_specs=[
            pl.BlockSpec((blk, 128, STIM), lambda i: (i, 0, 0)),
            pl.BlockSpec((1, STIM), lambda i: (0, 0)),
        ],
        out_specs=pl.BlockSpec((blk, 128), lambda i: (i, 0)),
        out_shape=jax.ShapeDtypeStruct((128, 128), jnp.float32),
    )(x3, wx)


def _tc_combine(stim, emb, b):
    def body(b_ref, s_ref, e_ref, o_ref):
        o_ref[...] = jax.nn.sigmoid(s_ref[...] + e_ref[...] + b_ref[0])

    return pl.pallas_call(
        body,
        in_specs=[
            pl.BlockSpec(memory_space=pltpu.SMEM),
            pl.BlockSpec((128, 128), lambda: (0, 0)),
            pl.BlockSpec((128, 128), lambda: (0, 0)),
        ],
        out_specs=pl.BlockSpec((128, 128), lambda: (0, 0)),
        out_shape=jax.ShapeDtypeStruct((128, 128), jnp.float32),
    )(b, stim, emb)


def kernel(X_stim, neuron_ids, embed_table, W, b):
    ids2d = neuron_ids.astype(jnp.int32).reshape(_N_CHUNKS, _CHUNK)
    table_t = jnp.transpose(embed_table)
    we = W[0, STIM:].reshape(EMB, 1)
    s2d = _tc_plane_sum(table_t, we)
    emb = _sc_row_pick(s2d, ids2d)
    x3 = X_stim.reshape(128, 128, STIM)
    wx = W[0, :STIM].reshape(1, STIM)
    stim = _tc_matvec(x3, wx)
    probs = _tc_combine(stim, emb, b)
    return probs.reshape(BATCH)
